# baseline (speedup 1.0000x reference)
import functools

import jax
import jax.numpy as jnp
from jax import lax
from jax.experimental import pallas as pl
from jax.experimental.pallas import tpu as pltpu
from jax.experimental.pallas import tpu_sc as plsc

F32 = jnp.float32
I32 = jnp.int32

H = 64
HH = 32
N = 50000
E = 800000
ND = 48
NP = N + ND
NSUB = 16
CH = 64
NCHUNK = 784
NRING = 4
EPT = CH * NCHUNK
EP = EPT * NSUB
ZROWS = NP // NSUB
RB = 5000


def _dot(a, b):
    dims = (((a.ndim - 1,), (0,)), ((), ()))

    def d(x, y):
        return jax.lax.dot_general(x, y, dims, preferred_element_type=F32)

    a_hi = a.astype(jnp.bfloat16).astype(F32)
    a_lo = a - a_hi
    b_hi = b.astype(jnp.bfloat16).astype(F32)
    b_lo = b - b_hi
    return d(a_hi, b_lo) + d(a_lo, b_hi) + d(a_hi, b_hi)


@functools.lru_cache(maxsize=None)
def _make_edge_kernel(with_counts, arow=0, drow=1):
    mesh = plsc.VectorSubcoreMesh(core_axis_name="c", subcore_axis_name="s")
    out_type = [jax.ShapeDtypeStruct((NP, HH), F32),
                jax.ShapeDtypeStruct((NP, HH), F32)]
    if with_counts:
        out_type += [jax.ShapeDtypeStruct((NP,), F32),
                     jax.ShapeDtypeStruct((NP,), F32)]
    scratch = [
        pltpu.VMEM_SHARED((NP, HH), F32),
    ]
    scratch += [pltpu.VMEM((3, CH), I32) for _ in range(NRING)]
    scratch += [pltpu.VMEM((CH, HH), F32) for _ in range(NRING)]
    scratch += [pltpu.VMEM((CH, HH), F32) for _ in range(NRING)]
    scratch += [
        pltpu.VMEM((HH,), F32),
        pltpu.VMEM((512,), F32),
        pltpu.SemaphoreType.DMA,
        pltpu.SemaphoreType.DMA,
        pltpu.SemaphoreType.DMA,
        pltpu.SemaphoreType.DMA,
    ]
    if with_counts:
        scratch += [
            pltpu.VMEM_SHARED((NP,), F32),
            pltpu.VMEM_SHARED((NP,), F32),
            pltpu.VMEM((CH,), F32),
        ]

    def body(a0_hbm, a1_hbm, b0_hbm, b1_hbm, w_hbm, comb_hbm, *rest):
        if with_counts:
            (s0_out, s1_out, n_dst_out, n_src_out, acc,
             c0, c1, c2, c3, ra0, ra1, ra2, ra3, rb0, rb1, rb2, rb3,
             wbuf, z1d, semi, semg0, semg1, sems,
             ndst, nsrc, onesb) = rest
        else:
            (s0_out, s1_out, acc,
             c0, c1, c2, c3, ra0, ra1, ra2, ra3, rb0, rb1, rb2, rb3,
             wbuf, z1d, semi, semg0, semg1, sems) = rest
        combs = [c0, c1, c2, c3]
        ras = [ra0, ra1, ra2, ra3]
        rbs = [rb0, rb1, rb2, rb3]
        semg = [semg0, semg1]

        cid = lax.axis_index("c")
        sid = lax.axis_index("s")
        zbase = sid * ZROWS
        zero16 = jnp.zeros((16,), F32)

        def zrow(j, _):
            ra0[j, pl.ds(0, 16)] = zero16
            ra0[j, pl.ds(16, 16)] = zero16
            return 0
        lax.fori_loop(0, CH, zrow, 0)
        zcps = []
        for k in range(ZROWS // CH):
            zcps.append(pltpu.async_copy(
                ra0, acc.at[pl.ds(zbase + k * CH, CH)], sems))
        ztail = ZROWS % CH
        if ztail:
            zcps.append(pltpu.async_copy(
                ra0.at[pl.ds(0, ztail)],
                acc.at[pl.ds(zbase + (ZROWS // CH) * CH, ztail)], sems))
        if with_counts:
            def z1(j, _):
                z1d[pl.ds(j * 16, 16)] = zero16
                return 0
            lax.fori_loop(0, 512 // 16, z1, 0)

            @pl.when(cid == 0)
            def _():
                for cnt in (ndst, nsrc):
                    for k in range(ZROWS // 512):
                        pltpu.async_copy(
                            z1d, cnt.at[pl.ds(zbase + k * 512, 512)],
                            sems).wait()
                    t1 = ZROWS % 512
                    if t1:
                        pltpu.async_copy(
                            z1d.at[pl.ds(0, t1)],
                            cnt.at[pl.ds(zbase + (ZROWS // 512) * 512, t1)],
                            sems).wait()
                ones16 = jnp.ones((16,), F32)
                for k in range(CH // 16):
                    onesb[pl.ds(k * 16, 16)] = ones16
        for cp in zcps:
            cp.wait()
        plsc.subcore_barrier()

        tile_chunk0 = sid * NCHUNK

        def run_half(a_hbm, b_hbm, wlo, do_counts):
            pltpu.sync_copy(w_hbm.at[pl.ds(wlo, HH)], wbuf)
            wv0 = wbuf[pl.ds(0, 16)]
            wv1 = wbuf[pl.ds(16, 16)]

            def comb_src(g):
                return comb_hbm.at[pl.ds((tile_chunk0 + g) * 3, 3)]

            def idx_fire(g, s):
                pltpu.async_copy(comb_src(g), combs[s], semi)

            def idx_wait(g, s):
                pltpu.make_async_copy(comb_src(g), combs[s], semi).wait()

            def gather_fire(s):
                sg = semg[s % 2]
                pltpu.async_copy(a_hbm.at[combs[s].at[arow]], ras[s], sg)
                pltpu.async_copy(b_hbm.at[combs[s].at[drow]], rbs[s], sg)

            def gather_wait(s):
                sg = semg[s % 2]
                pltpu.make_async_copy(
                    a_hbm.at[combs[s].at[arow]], ras[s], sg).wait()
                pltpu.make_async_copy(
                    b_hbm.at[combs[s].at[drow]], rbs[s], sg).wait()

            def scatter_fire(s):
                pltpu.async_copy(ras[s], acc.at[combs[s].at[drow]], sems,
                                 add=True)
                if do_counts:
                    pltpu.async_copy(onesb, ndst.at[combs[s].at[drow]], sems,
                                     add=True)
                    pltpu.async_copy(onesb, nsrc.at[combs[s].at[arow]], sems,
                                     add=True)

            def scatter_wait(s):
                pltpu.make_async_copy(
                    ras[s], acc.at[combs[s].at[drow]], sems).wait()
                if do_counts:
                    pltpu.make_async_copy(
                        onesb, ndst.at[combs[s].at[drow]], sems).wait()
                    pltpu.make_async_copy(
                        onesb, nsrc.at[combs[s].at[arow]], sems).wait()

            def compute(s):
                ra, rb, cb = ras[s], rbs[s], combs[s]

                def group(gi, _):
                    base = gi * 16
                    av16 = plsc.bitcast(cb[2, pl.ds(base, 16)], F32)
                    for k in range(16):
                        j = base + k
                        av = av16[k]
                        h0 = jnp.maximum(
                            ra[j, pl.ds(0, 16)] + rb[j, pl.ds(0, 16)]
                            + av * wv0, 0.0)
                        ra[j, pl.ds(0, 16)] = h0
                        h1 = jnp.maximum(
                            ra[j, pl.ds(16, 16)] + rb[j, pl.ds(16, 16)]
                            + av * wv1, 0.0)
                        ra[j, pl.ds(16, 16)] = h1
                    return 0
                lax.fori_loop(0, CH // 16, group, 0)

            pltpu.sync_copy(comb_src(0), c0)
            pltpu.sync_copy(comb_src(1), c1)
            idx_fire(2, 2)
            gather_fire(0)
            gather_fire(1)

            def outer(go, _):
                for p in range(NRING):
                    g = go * NRING + p
                    gather_wait(p)
                    compute(p)
                    if p == 0:
                        @pl.when(go > 0)
                        def _():
                            scatter_wait(NRING - 1)
                    else:
                        scatter_wait(p - 1)

                    @pl.when(g + 2 < NCHUNK)
                    def _():
                        idx_wait(g + 2, (p + 2) % NRING)

                    @pl.when(g + 3 < NCHUNK)
                    def _():
                        idx_fire(g + 3, (p + 3) % NRING)

                    @pl.when(g + 2 < NCHUNK)
                    def _():
                        gather_fire((p + 2) % NRING)
                    scatter_fire(p)
                return 0
            lax.fori_loop(0, NCHUNK // NRING, outer, 0)
            scatter_wait(NRING - 1)

        @pl.when(cid == 0)
        def _():
            run_half(a0_hbm, b0_hbm, 0, with_counts)

        @pl.when(cid == 1)
        def _():
            run_half(a1_hbm, b1_hbm, HH, False)

        plsc.subcore_barrier()

        sl = pl.ds(zbase, ZROWS)

        @pl.when(cid == 0)
        def _():
            pltpu.sync_copy(acc.at[sl], s0_out.at[sl])
            if with_counts:
                pltpu.sync_copy(ndst.at[sl], n_dst_out.at[sl])
                pltpu.sync_copy(nsrc.at[sl], n_src_out.at[sl])

        @pl.when(cid == 1)
        def _():
            pltpu.sync_copy(acc.at[sl], s1_out.at[sl])

    return pl.kernel(body, out_type=tuple(out_type), mesh=mesh,
                     scratch_types=scratch,
                     compiler_params=pltpu.CompilerParams(
                         use_tc_tiling_on_sc=False,
                         needs_layout_passes=False))


def _k_pre(xv_ref, xc_ref, wv_ref, bv_ref, wc_ref, bc_ref,
           g1s1_ref, g1d1_ref, g1b1_ref, g1d2_ref, g1b2_ref,
           v0_ref, c0_ref, a10_ref, a11_ref, b10_ref, b11_ref,
           b20_ref, b21_ref):
    v0 = jnp.maximum(_dot(xv_ref[...], wv_ref[...]) + bv_ref[...], 0.0)
    c0 = jnp.maximum(_dot(xc_ref[...], wc_ref[...]) + bc_ref[...], 0.0)
    v0_ref[...] = v0
    c0_ref[...] = c0
    a1 = _dot(v0, g1s1_ref[...])
    a10_ref[...] = a1[:, :HH]
    a11_ref[...] = a1[:, HH:]
    b1 = _dot(c0, g1d1_ref[...]) + g1b1_ref[...]
    b10_ref[...] = b1[:, :HH]
    b11_ref[...] = b1[:, HH:]
    b2 = _dot(v0, g1d2_ref[...]) + g1b2_ref[...]
    b20_ref[...] = b2[:, :HH]
    b21_ref[...] = b2[:, HH:]


def _k_post_a(s0_ref, s1_ref, n_ref, g2w_ref, g2b_ref,
              agg_ref, sum_ref, sq_ref):
    i = pl.program_id(0)
    agg = (_dot(s0_ref[...], g2w_ref[pl.ds(0, HH), :])
           + _dot(s1_ref[...], g2w_ref[pl.ds(HH, HH), :])
           + n_ref[...] * g2b_ref[...])
    agg_ref[...] = agg

    @pl.when(i == 0)
    def _():
        sum_ref[...] = jnp.zeros_like(sum_ref)
        sq_ref[...] = jnp.zeros_like(sq_ref)
    sum_ref[...] += jnp.sum(agg, axis=0, keepdims=True)
    sq_ref[...] += jnp.sum(agg * agg, axis=0, keepdims=True)


def _k_post1b(agg_ref, c0_ref, sum_ref, sq_ref, gam_ref, bet_ref,
              f1w_ref, f1b_ref, f2w_ref, f2b_ref, g1s2_ref,
              a20_ref, a21_ref):
    mean = sum_ref[...] * (1.0 / N)
    var = sq_ref[...] * (1.0 / N) - mean * mean
    inv = jax.lax.rsqrt(var + 1e-5)
    norm = (agg_ref[...] - mean) * inv * gam_ref[...] + bet_ref[...]
    x = (_dot(c0_ref[...], f1w_ref[pl.ds(0, H), :])
         + _dot(norm, f1w_ref[pl.ds(H, H), :]) + f1b_ref[...])
    h = jnp.maximum(x, 0.0)
    c1 = _dot(h, f2w_ref[...]) + f2b_ref[...]
    a2 = _dot(c1, g1s2_ref[...])
    a20_ref[...] = a2[:, :HH]
    a21_ref[...] = a2[:, HH:]


def _k_post2b(agg_ref, v0_ref, sum_ref, sq_ref, gam_ref, bet_ref,
              f1w_ref, f1b_ref, f2w_ref, f2b_ref, psum_ref):
    i = pl.program_id(0)
    mean = sum_ref[...] * (1.0 / N)
    var = sq_ref[...] * (1.0 / N) - mean * mean
    inv = jax.lax.rsqrt(var + 1e-5)
    norm = (agg_ref[...] - mean) * inv * gam_ref[...] + bet_ref[...]
    x = (_dot(v0_ref[...], f1w_ref[pl.ds(0, H), :])
         + _dot(norm, f1w_ref[pl.ds(H, H), :]) + f1b_ref[...])
    h = jnp.maximum(x, 0.0)
    v1 = _dot(h, f2w_ref[...]) + f2b_ref[...]

    @pl.when(i == 0)
    def _():
        psum_ref[...] = jnp.zeros_like(psum_ref)
    psum_ref[...] += jnp.sum(v1, axis=0, keepdims=True)


def _k_head(psum_ref, cv_ref, cd_ref, s1w_ref, s1b_ref, s2w_ref, s2b_ref,
            out_ref):
    pooled = psum_ref[...] * (1.0 / N)
    cv = cv_ref[0, 0]
    cd = cd_ref[0, 0]
    s1 = cv / (jnp.abs(cv) + 1.0)
    s2 = cd / (jnp.abs(cd) + 1.0)
    x = jnp.concatenate(
        [pooled, jnp.full((1, 1), s1, F32), jnp.full((1, 1), s2, F32)],
        axis=1)
    h = jnp.maximum(_dot(x, s1w_ref[...]) + s1b_ref[...], 0.0)
    out_ref[...] = _dot(h, s2w_ref[...]) + s2b_ref[...]


def _full_spec(arr):
    return pl.BlockSpec(arr.shape, lambda i: tuple(0 for _ in arr.shape))


def _rows_spec(width, rb=RB):
    return pl.BlockSpec((rb, width), lambda i: (i, 0))


def kernel(x_var, x_con, edge_index, edge_attr, cand_value, cand_depth,
           params):
    ei = edge_index.astype(I32)
    src1 = ei[1]
    dst1 = ei[0]
    attr = edge_attr[:, 0].astype(F32)

    pad = EP - E
    dump = N + (jnp.arange(pad, dtype=I32) % ND)
    attr_bits = jax.lax.bitcast_convert_type(
        jnp.concatenate([attr, jnp.zeros((pad,), F32)]), I32)

    sp = jnp.concatenate([src1, dump]).reshape(EP // CH, CH)
    dp = jnp.concatenate([dst1, dump]).reshape(EP // CH, CH)
    ab = attr_bits.reshape(EP // CH, CH)
    comb1 = jnp.stack([sp, dp, ab], axis=1).reshape(3 * EP // CH, CH)

    p = params
    wv, bv = p['var_init']
    wc, bc = p['con_init']
    pvc = p['v_to_c']
    pcv = p['c_to_v']
    g1w_1, g1b_1 = pvc['g1']
    g1w_2, g1b_2 = pcv['g1']
    g1s1, g1d1, g1a1 = g1w_1[:H], g1w_1[H:2 * H], g1w_1[2 * H]
    g1s2, g1d2, g1a2 = g1w_2[:H], g1w_2[H:2 * H], g1w_2[2 * H]

    grid10 = (N // RB,)

    pre_out = [jax.ShapeDtypeStruct((N, H), F32),
               jax.ShapeDtypeStruct((N, H), F32)]
    pre_out += [jax.ShapeDtypeStruct((NP, HH), F32)] * 6
    row64 = _rows_spec(H)
    row32 = _rows_spec(HH)
    RBP = 2000
    prow64 = _rows_spec(H, RBP)
    prow32 = _rows_spec(HH, RBP)
    (v0, c0, a10, a11, b10, b11, b20, b21) = pl.pallas_call(
        _k_pre,
        grid=(N // RBP,),
        in_specs=[_rows_spec(9, RBP), _rows_spec(5, RBP)]
        + [_full_spec(w) for w in
           (wv, bv.reshape(1, H), wc, bc.reshape(1, H), g1s1, g1d1,
            g1b_1.reshape(1, H), g1d2, g1b_2.reshape(1, H))],
        out_specs=[prow64, prow64, prow32, prow32, prow32, prow32, prow32,
                   prow32],
        out_shape=pre_out,
    )(x_var, x_con, wv, bv.reshape(1, H), wc, bc.reshape(1, H),
      g1s1, g1d1, g1b_1.reshape(1, H), g1d2, g1b_2.reshape(1, H))

    s10, s11, n1p, n2p = _make_edge_kernel(True)(
        a10, a11, b10, b11, g1a1, comb1)
    n1 = n1p[:N].reshape(N, 1)
    n2 = n2p[:N].reshape(N, 1)

    g2w_1, g2b_1 = pvc['g2']
    stat_spec = pl.BlockSpec((1, H), lambda i: (0, 0))
    agg1, sum1, sq1 = pl.pallas_call(
        _k_post_a,
        grid=grid10,
        in_specs=[row32, row32, pl.BlockSpec((RB, 1), lambda i: (i, 0)),
                  _full_spec(g2w_1), _full_spec(g2b_1.reshape(1, H))],
        out_specs=[row64, stat_spec, stat_spec],
        out_shape=[jax.ShapeDtypeStruct((N, H), F32),
                   jax.ShapeDtypeStruct((1, H), F32),
                   jax.ShapeDtypeStruct((1, H), F32)],
        compiler_params=pltpu.CompilerParams(
            dimension_semantics=("arbitrary",)),
    )(s10, s11, n1, g2w_1, g2b_1.reshape(1, H))

    f1w_1, f1b_1 = pvc['f1']
    f2w_1, f2b_1 = pvc['f2']
    a20, a21 = pl.pallas_call(
        _k_post1b,
        grid=grid10,
        in_specs=[row64, row64, stat_spec, stat_spec,
                  _full_spec(pvc['gamma'].reshape(1, H)),
                  _full_spec(pvc['beta'].reshape(1, H)),
                  _full_spec(f1w_1), _full_spec(f1b_1.reshape(1, H)),
                  _full_spec(f2w_1), _full_spec(f2b_1.reshape(1, H)),
                  _full_spec(g1s2)],
        out_specs=[row32, row32],
        out_shape=[jax.ShapeDtypeStruct((NP, HH), F32),
                   jax.ShapeDtypeStruct((NP, HH), F32)],
    )(agg1, c0, sum1, sq1, pvc['gamma'].reshape(1, H),
      pvc['beta'].reshape(1, H), f1w_1, f1b_1.reshape(1, H),
      f2w_1, f2b_1.reshape(1, H), g1s2)

    s20, s21 = _make_edge_kernel(False, arow=1, drow=0)(
        a20, a21, b20, b21, g1a2, comb1)

    g2w_2, g2b_2 = pcv['g2']
    agg2, sum2, sq2 = pl.pallas_call(
        _k_post_a,
        grid=grid10,
        in_specs=[row32, row32, pl.BlockSpec((RB, 1), lambda i: (i, 0)),
                  _full_spec(g2w_2), _full_spec(g2b_2.reshape(1, H))],
        out_specs=[row64, stat_spec, stat_spec],
        out_shape=[jax.ShapeDtypeStruct((N, H), F32),
                   jax.ShapeDtypeStruct((1, H), F32),
                   jax.ShapeDtypeStruct((1, H), F32)],
        compiler_params=pltpu.CompilerParams(
            dimension_semantics=("arbitrary",)),
    )(s20, s21, n2, g2w_2, g2b_2.reshape(1, H))

    f1w_2, f1b_2 = pcv['f1']
    f2w_2, f2b_2 = pcv['f2']
    (psum,) = pl.pallas_call(
        _k_post2b,
        grid=grid10,
        in_specs=[row64, row64, stat_spec, stat_spec,
                  _full_spec(pcv['gamma'].reshape(1, H)),
                  _full_spec(pcv['beta'].reshape(1, H)),
                  _full_spec(f1w_2), _full_spec(f1b_2.reshape(1, H)),
                  _full_spec(f2w_2), _full_spec(f2b_2.reshape(1, H))],
        out_specs=[stat_spec],
        out_shape=[jax.ShapeDtypeStruct((1, H), F32)],
        compiler_params=pltpu.CompilerParams(
            dimension_semantics=("arbitrary",)),
    )(agg2, v0, sum2, sq2, pcv['gamma'].reshape(1, H),
      pcv['beta'].reshape(1, H), f1w_2, f1b_2.reshape(1, H),
      f2w_2, f2b_2.reshape(1, H))

    s1w, s1b = p['score1']
    s2w, s2b = p['score2']
    out = pl.pallas_call(
        _k_head,
        grid=(1,),
        in_specs=[_full_spec(psum)] + [
            _full_spec(x) for x in
            (cand_value.reshape(1, 1), cand_depth.reshape(1, 1), s1w,
             s1b.reshape(1, H), s2w, s2b.reshape(1, 1))],
        out_specs=pl.BlockSpec((1, 1), lambda i: (0, 0)),
        out_shape=jax.ShapeDtypeStruct((1, 1), F32),
    )(psum, cand_value.reshape(1, 1), cand_depth.reshape(1, 1),
      s1w, s1b.reshape(1, H), s2w, s2b.reshape(1, 1))
    return out[0, 0]


# speedup vs baseline: 1.0340x; 1.0340x over previous
import functools

import jax
import jax.numpy as jnp
from jax import lax
from jax.experimental import pallas as pl
from jax.experimental.pallas import tpu as pltpu
from jax.experimental.pallas import tpu_sc as plsc

F32 = jnp.float32
I32 = jnp.int32

H = 64
HH = 32
N = 50000
E = 800000
ND = 48
NP = N + ND
NSUB = 16
CH = 64
NCHUNK = 784
NRING = 4
EPT = CH * NCHUNK
EP = EPT * NSUB
ZROWS = NP // NSUB
RB = 2000


def _dot(a, b):
    dims = (((a.ndim - 1,), (0,)), ((), ()))

    def d(x, y):
        return jax.lax.dot_general(x, y, dims, preferred_element_type=F32)

    a_hi = a.astype(jnp.bfloat16).astype(F32)
    a_lo = a - a_hi
    b_hi = b.astype(jnp.bfloat16).astype(F32)
    b_lo = b - b_hi
    return d(a_hi, b_lo) + d(a_lo, b_hi) + d(a_hi, b_hi)


@functools.lru_cache(maxsize=None)
def _make_edge_kernel(with_counts, arow=0, drow=1):
    mesh = plsc.VectorSubcoreMesh(core_axis_name="c", subcore_axis_name="s")
    out_type = [jax.ShapeDtypeStruct((NP, HH), F32),
                jax.ShapeDtypeStruct((NP, HH), F32)]
    if with_counts:
        out_type += [jax.ShapeDtypeStruct((NP,), F32),
                     jax.ShapeDtypeStruct((NP,), F32)]
    scratch = [
        pltpu.VMEM_SHARED((NP, HH), F32),
    ]
    scratch += [pltpu.VMEM((3, CH), I32) for _ in range(NRING)]
    scratch += [pltpu.VMEM((CH, HH), F32) for _ in range(NRING)]
    scratch += [pltpu.VMEM((CH, HH), F32) for _ in range(NRING)]
    scratch += [
        pltpu.VMEM((HH,), F32),
        pltpu.VMEM((512,), F32),
        pltpu.SemaphoreType.DMA,
        pltpu.SemaphoreType.DMA,
        pltpu.SemaphoreType.DMA,
        pltpu.SemaphoreType.DMA,
    ]
    if with_counts:
        scratch += [
            pltpu.VMEM_SHARED((NP,), F32),
            pltpu.VMEM_SHARED((NP,), F32),
            pltpu.VMEM((CH,), F32),
        ]

    def body(a0_hbm, a1_hbm, b0_hbm, b1_hbm, w_hbm, comb_hbm, *rest):
        if with_counts:
            (s0_out, s1_out, n_dst_out, n_src_out, acc,
             c0, c1, c2, c3, ra0, ra1, ra2, ra3, rb0, rb1, rb2, rb3,
             wbuf, z1d, semi, semg0, semg1, sems,
             ndst, nsrc, onesb) = rest
        else:
            (s0_out, s1_out, acc,
             c0, c1, c2, c3, ra0, ra1, ra2, ra3, rb0, rb1, rb2, rb3,
             wbuf, z1d, semi, semg0, semg1, sems) = rest
        combs = [c0, c1, c2, c3]
        ras = [ra0, ra1, ra2, ra3]
        rbs = [rb0, rb1, rb2, rb3]
        semg = [semg0, semg1]

        cid = lax.axis_index("c")
        sid = lax.axis_index("s")
        zbase = sid * ZROWS
        zero16 = jnp.zeros((16,), F32)

        def zrow(j, _):
            ra0[j, pl.ds(0, 16)] = zero16
            ra0[j, pl.ds(16, 16)] = zero16
            return 0
        lax.fori_loop(0, CH, zrow, 0)
        zcps = []
        for k in range(ZROWS // CH):
            zcps.append(pltpu.async_copy(
                ra0, acc.at[pl.ds(zbase + k * CH, CH)], sems))
        ztail = ZROWS % CH
        if ztail:
            zcps.append(pltpu.async_copy(
                ra0.at[pl.ds(0, ztail)],
                acc.at[pl.ds(zbase + (ZROWS // CH) * CH, ztail)], sems))
        if with_counts:
            def z1(j, _):
                z1d[pl.ds(j * 16, 16)] = zero16
                return 0
            lax.fori_loop(0, 512 // 16, z1, 0)

            @pl.when(cid == 0)
            def _():
                for cnt in (ndst, nsrc):
                    for k in range(ZROWS // 512):
                        pltpu.async_copy(
                            z1d, cnt.at[pl.ds(zbase + k * 512, 512)],
                            sems).wait()
                    t1 = ZROWS % 512
                    if t1:
                        pltpu.async_copy(
                            z1d.at[pl.ds(0, t1)],
                            cnt.at[pl.ds(zbase + (ZROWS // 512) * 512, t1)],
                            sems).wait()
                ones16 = jnp.ones((16,), F32)
                for k in range(CH // 16):
                    onesb[pl.ds(k * 16, 16)] = ones16
        for cp in zcps:
            cp.wait()
        plsc.subcore_barrier()

        tile_chunk0 = sid * NCHUNK

        def run_half(a_hbm, b_hbm, wlo, do_counts):
            pltpu.sync_copy(w_hbm.at[pl.ds(wlo, HH)], wbuf)
            wv0 = wbuf[pl.ds(0, 16)]
            wv1 = wbuf[pl.ds(16, 16)]

            def comb_src(g):
                return comb_hbm.at[pl.ds((tile_chunk0 + g) * 3, 3)]

            def idx_fire(g, s):
                pltpu.async_copy(comb_src(g), combs[s], semi)

            def idx_wait(g, s):
                pltpu.make_async_copy(comb_src(g), combs[s], semi).wait()

            def gather_fire(s):
                sg = semg[s % 2]
                pltpu.async_copy(a_hbm.at[combs[s].at[arow]], ras[s], sg)
                pltpu.async_copy(b_hbm.at[combs[s].at[drow]], rbs[s], sg)

            def gather_wait(s):
                sg = semg[s % 2]
                pltpu.make_async_copy(
                    a_hbm.at[combs[s].at[arow]], ras[s], sg).wait()
                pltpu.make_async_copy(
                    b_hbm.at[combs[s].at[drow]], rbs[s], sg).wait()

            def scatter_fire(s):
                pltpu.async_copy(ras[s], acc.at[combs[s].at[drow]], sems,
                                 add=True)
                if do_counts:
                    pltpu.async_copy(onesb, ndst.at[combs[s].at[drow]], sems,
                                     add=True)
                    pltpu.async_copy(onesb, nsrc.at[combs[s].at[arow]], sems,
                                     add=True)

            def scatter_wait(s):
                pltpu.make_async_copy(
                    ras[s], acc.at[combs[s].at[drow]], sems).wait()
                if do_counts:
                    pltpu.make_async_copy(
                        onesb, ndst.at[combs[s].at[drow]], sems).wait()
                    pltpu.make_async_copy(
                        onesb, nsrc.at[combs[s].at[arow]], sems).wait()

            def compute(s):
                ra, rb, cb = ras[s], rbs[s], combs[s]

                def group(gi, _):
                    base = gi * 16
                    av16 = plsc.bitcast(cb[2, pl.ds(base, 16)], F32)
                    for k in range(16):
                        j = base + k
                        av = av16[k]
                        h0 = jnp.maximum(
                            ra[j, pl.ds(0, 16)] + rb[j, pl.ds(0, 16)]
                            + av * wv0, 0.0)
                        ra[j, pl.ds(0, 16)] = h0
                        h1 = jnp.maximum(
                            ra[j, pl.ds(16, 16)] + rb[j, pl.ds(16, 16)]
                            + av * wv1, 0.0)
                        ra[j, pl.ds(16, 16)] = h1
                    return 0
                lax.fori_loop(0, CH // 16, group, 0)

            pltpu.sync_copy(comb_src(0), c0)
            pltpu.sync_copy(comb_src(1), c1)
            idx_fire(2, 2)
            gather_fire(0)
            gather_fire(1)

            def outer(go, _):
                for p in range(NRING):
                    g = go * NRING + p
                    gather_wait(p)
                    compute(p)
                    if p == 0:
                        @pl.when(go > 0)
                        def _():
                            scatter_wait(NRING - 1)
                    else:
                        scatter_wait(p - 1)

                    @pl.when(g + 2 < NCHUNK)
                    def _():
                        idx_wait(g + 2, (p + 2) % NRING)

                    @pl.when(g + 3 < NCHUNK)
                    def _():
                        idx_fire(g + 3, (p + 3) % NRING)

                    @pl.when(g + 2 < NCHUNK)
                    def _():
                        gather_fire((p + 2) % NRING)
                    scatter_fire(p)
                return 0
            lax.fori_loop(0, NCHUNK // NRING, outer, 0)
            scatter_wait(NRING - 1)

        @pl.when(cid == 0)
        def _():
            run_half(a0_hbm, b0_hbm, 0, with_counts)

        @pl.when(cid == 1)
        def _():
            run_half(a1_hbm, b1_hbm, HH, False)

        plsc.subcore_barrier()

        sl = pl.ds(zbase, ZROWS)

        @pl.when(cid == 0)
        def _():
            pltpu.sync_copy(acc.at[sl], s0_out.at[sl])
            if with_counts:
                pltpu.sync_copy(ndst.at[sl], n_dst_out.at[sl])
                pltpu.sync_copy(nsrc.at[sl], n_src_out.at[sl])

        @pl.when(cid == 1)
        def _():
            pltpu.sync_copy(acc.at[sl], s1_out.at[sl])

    return pl.kernel(body, out_type=tuple(out_type), mesh=mesh,
                     scratch_types=scratch,
                     compiler_params=pltpu.CompilerParams(
                         use_tc_tiling_on_sc=False,
                         needs_layout_passes=False))


def _k_pre(xv_ref, xc_ref, wv_ref, bv_ref, wc_ref, bc_ref,
           g1s1_ref, g1d1_ref, g1b1_ref, g1d2_ref, g1b2_ref,
           v0_ref, c0_ref, a10_ref, a11_ref, b10_ref, b11_ref,
           b20_ref, b21_ref):
    v0 = jnp.maximum(_dot(xv_ref[...], wv_ref[...]) + bv_ref[...], 0.0)
    c0 = jnp.maximum(_dot(xc_ref[...], wc_ref[...]) + bc_ref[...], 0.0)
    v0_ref[...] = v0
    c0_ref[...] = c0
    a1 = _dot(v0, g1s1_ref[...])
    a10_ref[...] = a1[:, :HH]
    a11_ref[...] = a1[:, HH:]
    b1 = _dot(c0, g1d1_ref[...]) + g1b1_ref[...]
    b10_ref[...] = b1[:, :HH]
    b11_ref[...] = b1[:, HH:]
    b2 = _dot(v0, g1d2_ref[...]) + g1b2_ref[...]
    b20_ref[...] = b2[:, :HH]
    b21_ref[...] = b2[:, HH:]


def _k_post_a(s0_ref, s1_ref, n_ref, g2w_ref, g2b_ref,
              agg_ref, sum_ref, sq_ref):
    i = pl.program_id(0)
    agg = (_dot(s0_ref[...], g2w_ref[pl.ds(0, HH), :])
           + _dot(s1_ref[...], g2w_ref[pl.ds(HH, HH), :])
           + n_ref[...] * g2b_ref[...])
    agg_ref[...] = agg

    @pl.when(i == 0)
    def _():
        sum_ref[...] = jnp.zeros_like(sum_ref)
        sq_ref[...] = jnp.zeros_like(sq_ref)
    sum_ref[...] += jnp.sum(agg, axis=0, keepdims=True)
    sq_ref[...] += jnp.sum(agg * agg, axis=0, keepdims=True)


def _k_post1b(agg_ref, c0_ref, sum_ref, sq_ref, gam_ref, bet_ref,
              f1w_ref, f1b_ref, f2w_ref, f2b_ref, g1s2_ref,
              a20_ref, a21_ref):
    mean = sum_ref[...] * (1.0 / N)
    var = sq_ref[...] * (1.0 / N) - mean * mean
    inv = jax.lax.rsqrt(var + 1e-5)
    norm = (agg_ref[...] - mean) * inv * gam_ref[...] + bet_ref[...]
    x = (_dot(c0_ref[...], f1w_ref[pl.ds(0, H), :])
         + _dot(norm, f1w_ref[pl.ds(H, H), :]) + f1b_ref[...])
    h = jnp.maximum(x, 0.0)
    c1 = _dot(h, f2w_ref[...]) + f2b_ref[...]
    a2 = _dot(c1, g1s2_ref[...])
    a20_ref[...] = a2[:, :HH]
    a21_ref[...] = a2[:, HH:]


def _k_post2b(agg_ref, v0_ref, sum_ref, sq_ref, gam_ref, bet_ref,
              f1w_ref, f1b_ref, f2w_ref, f2b_ref, psum_ref):
    i = pl.program_id(0)
    mean = sum_ref[...] * (1.0 / N)
    var = sq_ref[...] * (1.0 / N) - mean * mean
    inv = jax.lax.rsqrt(var + 1e-5)
    norm = (agg_ref[...] - mean) * inv * gam_ref[...] + bet_ref[...]
    x = (_dot(v0_ref[...], f1w_ref[pl.ds(0, H), :])
         + _dot(norm, f1w_ref[pl.ds(H, H), :]) + f1b_ref[...])
    h = jnp.maximum(x, 0.0)
    v1 = _dot(h, f2w_ref[...]) + f2b_ref[...]

    @pl.when(i == 0)
    def _():
        psum_ref[...] = jnp.zeros_like(psum_ref)
    psum_ref[...] += jnp.sum(v1, axis=0, keepdims=True)


def _k_head(psum_ref, cv_ref, cd_ref, s1w_ref, s1b_ref, s2w_ref, s2b_ref,
            out_ref):
    pooled = psum_ref[...] * (1.0 / N)
    cv = cv_ref[0, 0]
    cd = cd_ref[0, 0]
    s1 = cv / (jnp.abs(cv) + 1.0)
    s2 = cd / (jnp.abs(cd) + 1.0)
    x = jnp.concatenate(
        [pooled, jnp.full((1, 1), s1, F32), jnp.full((1, 1), s2, F32)],
        axis=1)
    h = jnp.maximum(_dot(x, s1w_ref[...]) + s1b_ref[...], 0.0)
    out_ref[...] = _dot(h, s2w_ref[...]) + s2b_ref[...]


def _full_spec(arr):
    return pl.BlockSpec(arr.shape, lambda i: tuple(0 for _ in arr.shape))


def _rows_spec(width, rb=RB):
    return pl.BlockSpec((rb, width), lambda i: (i, 0))


def kernel(x_var, x_con, edge_index, edge_attr, cand_value, cand_depth,
           params):
    ei = edge_index.astype(I32)
    src1 = ei[1]
    dst1 = ei[0]
    attr = edge_attr[:, 0].astype(F32)

    pad = EP - E
    dump = N + (jnp.arange(pad, dtype=I32) % ND)
    attr_bits = jax.lax.bitcast_convert_type(
        jnp.concatenate([attr, jnp.zeros((pad,), F32)]), I32)

    sp = jnp.concatenate([src1, dump]).reshape(EP // CH, CH)
    dp = jnp.concatenate([dst1, dump]).reshape(EP // CH, CH)
    ab = attr_bits.reshape(EP // CH, CH)
    comb1 = jnp.stack([sp, dp, ab], axis=1).reshape(3 * EP // CH, CH)

    p = params
    wv, bv = p['var_init']
    wc, bc = p['con_init']
    pvc = p['v_to_c']
    pcv = p['c_to_v']
    g1w_1, g1b_1 = pvc['g1']
    g1w_2, g1b_2 = pcv['g1']
    g1s1, g1d1, g1a1 = g1w_1[:H], g1w_1[H:2 * H], g1w_1[2 * H]
    g1s2, g1d2, g1a2 = g1w_2[:H], g1w_2[H:2 * H], g1w_2[2 * H]

    grid10 = (N // RB,)

    pre_out = [jax.ShapeDtypeStruct((N, H), F32),
               jax.ShapeDtypeStruct((N, H), F32)]
    pre_out += [jax.ShapeDtypeStruct((NP, HH), F32)] * 6
    row64 = _rows_spec(H)
    row32 = _rows_spec(HH)
    RBP = 2000
    prow64 = _rows_spec(H, RBP)
    prow32 = _rows_spec(HH, RBP)
    (v0, c0, a10, a11, b10, b11, b20, b21) = pl.pallas_call(
        _k_pre,
        grid=(N // RBP,),
        in_specs=[_rows_spec(9, RBP), _rows_spec(5, RBP)]
        + [_full_spec(w) for w in
           (wv, bv.reshape(1, H), wc, bc.reshape(1, H), g1s1, g1d1,
            g1b_1.reshape(1, H), g1d2, g1b_2.reshape(1, H))],
        out_specs=[prow64, prow64, prow32, prow32, prow32, prow32, prow32,
                   prow32],
        out_shape=pre_out,
    )(x_var, x_con, wv, bv.reshape(1, H), wc, bc.reshape(1, H),
      g1s1, g1d1, g1b_1.reshape(1, H), g1d2, g1b_2.reshape(1, H))

    s10, s11, n1p, n2p = _make_edge_kernel(True)(
        a10, a11, b10, b11, g1a1, comb1)
    n1 = n1p[:N].reshape(N, 1)
    n2 = n2p[:N].reshape(N, 1)

    g2w_1, g2b_1 = pvc['g2']
    stat_spec = pl.BlockSpec((1, H), lambda i: (0, 0))
    agg1, sum1, sq1 = pl.pallas_call(
        _k_post_a,
        grid=grid10,
        in_specs=[row32, row32, pl.BlockSpec((RB, 1), lambda i: (i, 0)),
                  _full_spec(g2w_1), _full_spec(g2b_1.reshape(1, H))],
        out_specs=[row64, stat_spec, stat_spec],
        out_shape=[jax.ShapeDtypeStruct((N, H), F32),
                   jax.ShapeDtypeStruct((1, H), F32),
                   jax.ShapeDtypeStruct((1, H), F32)],
        compiler_params=pltpu.CompilerParams(
            dimension_semantics=("arbitrary",)),
    )(s10, s11, n1, g2w_1, g2b_1.reshape(1, H))

    f1w_1, f1b_1 = pvc['f1']
    f2w_1, f2b_1 = pvc['f2']
    a20, a21 = pl.pallas_call(
        _k_post1b,
        grid=grid10,
        in_specs=[row64, row64, stat_spec, stat_spec,
                  _full_spec(pvc['gamma'].reshape(1, H)),
                  _full_spec(pvc['beta'].reshape(1, H)),
                  _full_spec(f1w_1), _full_spec(f1b_1.reshape(1, H)),
                  _full_spec(f2w_1), _full_spec(f2b_1.reshape(1, H)),
                  _full_spec(g1s2)],
        out_specs=[row32, row32],
        out_shape=[jax.ShapeDtypeStruct((NP, HH), F32),
                   jax.ShapeDtypeStruct((NP, HH), F32)],
    )(agg1, c0, sum1, sq1, pvc['gamma'].reshape(1, H),
      pvc['beta'].reshape(1, H), f1w_1, f1b_1.reshape(1, H),
      f2w_1, f2b_1.reshape(1, H), g1s2)

    s20, s21 = _make_edge_kernel(False, arow=1, drow=0)(
        a20, a21, b20, b21, g1a2, comb1)

    g2w_2, g2b_2 = pcv['g2']
    agg2, sum2, sq2 = pl.pallas_call(
        _k_post_a,
        grid=grid10,
        in_specs=[row32, row32, pl.BlockSpec((RB, 1), lambda i: (i, 0)),
                  _full_spec(g2w_2), _full_spec(g2b_2.reshape(1, H))],
        out_specs=[row64, stat_spec, stat_spec],
        out_shape=[jax.ShapeDtypeStruct((N, H), F32),
                   jax.ShapeDtypeStruct((1, H), F32),
                   jax.ShapeDtypeStruct((1, H), F32)],
        compiler_params=pltpu.CompilerParams(
            dimension_semantics=("arbitrary",)),
    )(s20, s21, n2, g2w_2, g2b_2.reshape(1, H))

    f1w_2, f1b_2 = pcv['f1']
    f2w_2, f2b_2 = pcv['f2']
    (psum,) = pl.pallas_call(
        _k_post2b,
        grid=grid10,
        in_specs=[row64, row64, stat_spec, stat_spec,
                  _full_spec(pcv['gamma'].reshape(1, H)),
                  _full_spec(pcv['beta'].reshape(1, H)),
                  _full_spec(f1w_2), _full_spec(f1b_2.reshape(1, H)),
                  _full_spec(f2w_2), _full_spec(f2b_2.reshape(1, H))],
        out_specs=[stat_spec],
        out_shape=[jax.ShapeDtypeStruct((1, H), F32)],
        compiler_params=pltpu.CompilerParams(
            dimension_semantics=("arbitrary",)),
    )(agg2, v0, sum2, sq2, pcv['gamma'].reshape(1, H),
      pcv['beta'].reshape(1, H), f1w_2, f1b_2.reshape(1, H),
      f2w_2, f2b_2.reshape(1, H))

    s1w, s1b = p['score1']
    s2w, s2b = p['score2']
    out = pl.pallas_call(
        _k_head,
        grid=(1,),
        in_specs=[_full_spec(psum)] + [
            _full_spec(x) for x in
            (cand_value.reshape(1, 1), cand_depth.reshape(1, 1), s1w,
             s1b.reshape(1, H), s2w, s2b.reshape(1, 1))],
        out_specs=pl.BlockSpec((1, 1), lambda i: (0, 0)),
        out_shape=jax.ShapeDtypeStruct((1, 1), F32),
    )(psum, cand_value.reshape(1, 1), cand_depth.reshape(1, 1),
      s1w, s1b.reshape(1, H), s2w, s2b.reshape(1, 1))
    return out[0, 0]
